# Initial kernel scaffold; baseline (speedup 1.0000x reference)
import jax
import jax.numpy as jnp
from jax.experimental import pallas as pl


def kernel(x, edge_index, W1_0, b1_0, W2_0, b2_0, W1_1, b1_1, W2_1, b2_1, W1_2, b1_2, W2_2, b2_2, W_jk, b_jk, Wc1, bc1, bn_gamma, bn_beta, bn_mean, bn_var, Wc2, bc2):
    raise NotImplementedError("write your pallas kernel here")


# speedup vs baseline: 5.5272x; 5.5272x over previous
import functools

import jax
import jax.numpy as jnp
from jax import lax
from jax.experimental import pallas as pl
from jax.experimental.pallas import tpu as pltpu
from jax.experimental.pallas import tpu_sc as plsc

N = 10000
E = 320000
H = 128
BN_EPS = 1e-5

NC = 2
NS = 16
NW = NC * NS
EPT = E // NW
CH = 128
NFULL = EPT // CH
REM = EPT - NFULL * CH
N_PAD = 10240
ROWS_PT = N_PAD // NS


def _agg_body(h_hbm, src_hbm, dst_hbm, zeros_hbm, out_hbm,
              acc_sh, src_v, dst_v, rows_v, src_r, dst_r, rows_r, sem):
    cid = lax.axis_index("c")
    sid = lax.axis_index("s")
    pltpu.sync_copy(zeros_hbm.at[pl.ds(sid * ROWS_PT, ROWS_PT)],
                    acc_sh.at[pl.ds(sid * ROWS_PT, ROWS_PT)])
    plsc.subcore_barrier()

    wid = sid * NC + cid
    base = wid * EPT

    def chunk(i, carry):
        off = base + i * CH
        pltpu.sync_copy(src_hbm.at[pl.ds(off, CH)], src_v)
        pltpu.sync_copy(dst_hbm.at[pl.ds(off, CH)], dst_v)
        pltpu.async_copy(h_hbm.at[src_v], rows_v, sem).wait()
        pltpu.sync_copy(rows_v, acc_sh.at[dst_v], add=True)
        return carry

    lax.fori_loop(0, NFULL, chunk, 0)

    off = base + NFULL * CH
    pltpu.sync_copy(src_hbm.at[pl.ds(off, REM)], src_r)
    pltpu.sync_copy(dst_hbm.at[pl.ds(off, REM)], dst_r)
    pltpu.async_copy(h_hbm.at[src_r], rows_r, sem).wait()
    pltpu.sync_copy(rows_r, acc_sh.at[dst_r], add=True)

    plsc.subcore_barrier()
    pltpu.sync_copy(acc_sh.at[pl.ds(sid * ROWS_PT, ROWS_PT)],
                    out_hbm.at[cid, pl.ds(sid * ROWS_PT, ROWS_PT)])


_agg_call = pl.kernel(
    _agg_body,
    out_type=jax.ShapeDtypeStruct((NC, N_PAD, H), jnp.float32),
    mesh=plsc.VectorSubcoreMesh(core_axis_name="c", subcore_axis_name="s",
                                num_cores=NC, num_subcores=NS),
    scratch_types=[
        pltpu.VMEM_SHARED((N_PAD, H), jnp.float32),
        pltpu.VMEM((CH,), jnp.int32),
        pltpu.VMEM((CH,), jnp.int32),
        pltpu.VMEM((CH, H), jnp.float32),
        pltpu.VMEM((REM,), jnp.int32),
        pltpu.VMEM((REM,), jnp.int32),
        pltpu.VMEM((REM, H), jnp.float32),
        pltpu.SemaphoreType.DMA,
    ],
)


BLK = 2000


def _layer_body(h_ref, p0_ref, p1_ref, W1_ref, b1_ref, W2_ref, b2_ref, o_ref):
    z = h_ref[...] + p0_ref[...] + p1_ref[...]
    a = jnp.dot(z, W1_ref[...], preferred_element_type=jnp.float32)
    a = jnp.maximum(a + b1_ref[...], 0.0)
    o = jnp.dot(a, W2_ref[...], preferred_element_type=jnp.float32)
    o_ref[...] = jnp.maximum(o + b2_ref[...], 0.0)


def _mlp_layer(h, p0, p1, W1, b1, W2, b2):
    row = pl.BlockSpec((BLK, H), lambda i: (i, 0))
    full = pl.BlockSpec((H, H), lambda i: (0, 0))
    vec = pl.BlockSpec((1, H), lambda i: (0, 0))
    return pl.pallas_call(
        _layer_body,
        grid=(N // BLK,),
        in_specs=[row, row, row, full, vec, full, vec],
        out_specs=row,
        out_shape=jax.ShapeDtypeStruct((N, H), jnp.float32),
    )(h, p0, p1, W1, b1.reshape(1, H), W2, b2.reshape(1, H))


def _final_body(h1_ref, h2_ref, h3_ref, Wj1_ref, Wj2_ref, Wj3_ref, bjk_ref,
                Wc1_ref, bc1_ref, g_ref, b_ref, m_ref, v_ref, Wc2_ref, bc2_ref,
                o_ref):
    t = jnp.dot(h1_ref[...], Wj1_ref[...], preferred_element_type=jnp.float32)
    t += jnp.dot(h2_ref[...], Wj2_ref[...], preferred_element_type=jnp.float32)
    t += jnp.dot(h3_ref[...], Wj3_ref[...], preferred_element_type=jnp.float32)
    t += bjk_ref[...]
    u = jnp.dot(t, Wc1_ref[...], preferred_element_type=jnp.float32)
    u = u + bc1_ref[...]
    u = (u - m_ref[...]) / jnp.sqrt(v_ref[...] + BN_EPS) * g_ref[...] + b_ref[...]
    u = jnp.maximum(u, 0.0)
    o = jnp.dot(u, Wc2_ref[...], preferred_element_type=jnp.float32)
    o_ref[...] = o + bc2_ref[...]


def _final(h1, h2, h3, W_jk, b_jk, Wc1, bc1, g, b, m, v, Wc2, bc2):
    row = pl.BlockSpec((BLK, H), lambda i: (i, 0))
    full = pl.BlockSpec((H, H), lambda i: (0, 0))
    vec = pl.BlockSpec((1, H), lambda i: (0, 0))
    return pl.pallas_call(
        _final_body,
        grid=(N // BLK,),
        in_specs=[row, row, row, full, full, full, vec, full, vec,
                  vec, vec, vec, vec, full, vec],
        out_specs=row,
        out_shape=jax.ShapeDtypeStruct((N, H), jnp.float32),
    )(h1, h2, h3, W_jk[0:H], W_jk[H:2 * H], W_jk[2 * H:3 * H],
      b_jk.reshape(1, H), Wc1, bc1.reshape(1, H), g.reshape(1, H),
      b.reshape(1, H), m.reshape(1, H), v.reshape(1, H), Wc2,
      bc2.reshape(1, H))


def kernel(x, edge_index, W1_0, b1_0, W2_0, b2_0, W1_1, b1_1, W2_1, b2_1,
           W1_2, b1_2, W2_2, b2_2, W_jk, b_jk, Wc1, bc1, bn_gamma, bn_beta,
           bn_mean, bn_var, Wc2, bc2):
    src = edge_index[0]
    dst = edge_index[1]
    zeros = jnp.zeros((N_PAD, H), jnp.float32)
    Ws = [(W1_0, b1_0, W2_0, b2_0), (W1_1, b1_1, W2_1, b2_1),
          (W1_2, b1_2, W2_2, b2_2)]
    h = x
    xs = []
    for (W1, b1, W2, b2) in Ws:
        parts = _agg_call(h, src, dst, zeros)
        h = _mlp_layer(h, parts[0, :N], parts[1, :N], W1, b1, W2, b2)
        xs.append(h)
    return _final(xs[0], xs[1], xs[2], W_jk, b_jk, Wc1, bc1, bn_gamma,
                  bn_beta, bn_mean, bn_var, Wc2, bc2)


# speedup vs baseline: 10.9113x; 1.9741x over previous
import functools

import jax
import jax.numpy as jnp
from jax import lax
from jax.experimental import pallas as pl
from jax.experimental.pallas import tpu as pltpu
from jax.experimental.pallas import tpu_sc as plsc

N = 10000
E = 320000
H = 128
BN_EPS = 1e-5

NC = 2
NS = 16
NW = NC * NS
CH = 128
NCH = 80
E_PAD = NW * NCH * CH
N_PAD = 10240
ROWS_PT = N_PAD // NS
NBUF = 2


G = 40


def _agg_body(h_hbm, src_hbm, dst_hbm, zeros_hbm, out_hbm,
              acc_sh, src_g, dst_g, b0, b1, sem_z, sem_i, s0, s1):
    cid = lax.axis_index("c")
    sid = lax.axis_index("s")
    wid = sid * NC + cid
    crow = wid * NCH
    bufs = (b0, b1)
    sems = (s0, s1)

    zcp = pltpu.make_async_copy(
        zeros_hbm.at[pl.ds(sid * ROWS_PT, ROWS_PT)],
        acc_sh.at[pl.ds(sid * ROWS_PT, ROWS_PT)], sem_z)
    zcp.start()

    def load_group(g):
        pltpu.sync_copy(src_hbm.at[pl.ds(crow + g * G, G)], src_g)
        pltpu.sync_copy(dst_hbm.at[pl.ds(crow + g * G, G)], dst_g)

    def fire(k, b):
        pltpu.make_async_copy(h_hbm.at[src_g.at[k]], bufs[b], sems[b]).start()

    load_group(0)
    fire(0, 0)
    fire(1, 1)
    zcp.wait()
    plsc.subcore_barrier()

    for g in range(NCH // G):
        if g > 0:
            load_group(g)
            fire(0, 0)
            fire(1, 1)

        def step(j, carry):
            for b in range(NBUF):
                k = j * NBUF + b
                pltpu.make_async_copy(h_hbm.at[src_g.at[k]], bufs[b],
                                      sems[b]).wait()
                pltpu.sync_copy(bufs[b], acc_sh.at[dst_g.at[k]], add=True)

                @pl.when(k + NBUF < G)
                def _():
                    pltpu.make_async_copy(h_hbm.at[src_g.at[k + NBUF]],
                                          bufs[b], sems[b]).start()
            return carry

        lax.fori_loop(0, G // NBUF, step, 0)

    plsc.subcore_barrier()
    pltpu.sync_copy(acc_sh.at[pl.ds(sid * ROWS_PT, ROWS_PT)],
                    out_hbm.at[cid, pl.ds(sid * ROWS_PT, ROWS_PT)])


_agg_call = pl.kernel(
    _agg_body,
    out_type=jax.ShapeDtypeStruct((NC, N_PAD, H), jnp.float32),
    mesh=plsc.VectorSubcoreMesh(core_axis_name="c", subcore_axis_name="s",
                                num_cores=NC, num_subcores=NS),
    scratch_types=[
        pltpu.VMEM_SHARED((N_PAD, H), jnp.float32),
        pltpu.VMEM((G, CH), jnp.int32),
        pltpu.VMEM((G, CH), jnp.int32),
        pltpu.VMEM((CH, H), jnp.float32),
        pltpu.VMEM((CH, H), jnp.float32),
        pltpu.SemaphoreType.DMA,
        pltpu.SemaphoreType.DMA,
        pltpu.SemaphoreType.DMA,
        pltpu.SemaphoreType.DMA,
    ],
)


BLK = 2000


def _layer_body(h_ref, p0_ref, p1_ref, W1_ref, b1_ref, W2_ref, b2_ref, o_ref):
    z = h_ref[...] + p0_ref[...] + p1_ref[...]
    a = jnp.dot(z, W1_ref[...], preferred_element_type=jnp.float32)
    a = jnp.maximum(a + b1_ref[...], 0.0)
    o = jnp.dot(a, W2_ref[...], preferred_element_type=jnp.float32)
    o_ref[...] = jnp.maximum(o + b2_ref[...], 0.0)


def _mlp_layer(h, p0, p1, W1, b1, W2, b2):
    row = pl.BlockSpec((BLK, H), lambda i: (i, 0))
    full = pl.BlockSpec((H, H), lambda i: (0, 0))
    vec = pl.BlockSpec((1, H), lambda i: (0, 0))
    return pl.pallas_call(
        _layer_body,
        grid=(N // BLK,),
        in_specs=[row, row, row, full, vec, full, vec],
        out_specs=row,
        out_shape=jax.ShapeDtypeStruct((N, H), jnp.float32),
    )(h, p0, p1, W1, b1.reshape(1, H), W2, b2.reshape(1, H))


def _final_body(h1_ref, h2_ref, h3_ref, Wj1_ref, Wj2_ref, Wj3_ref, bjk_ref,
                Wc1_ref, bc1_ref, g_ref, b_ref, m_ref, v_ref, Wc2_ref, bc2_ref,
                o_ref):
    t = jnp.dot(h1_ref[...], Wj1_ref[...], preferred_element_type=jnp.float32)
    t += jnp.dot(h2_ref[...], Wj2_ref[...], preferred_element_type=jnp.float32)
    t += jnp.dot(h3_ref[...], Wj3_ref[...], preferred_element_type=jnp.float32)
    t += bjk_ref[...]
    u = jnp.dot(t, Wc1_ref[...], preferred_element_type=jnp.float32)
    u = u + bc1_ref[...]
    u = (u - m_ref[...]) / jnp.sqrt(v_ref[...] + BN_EPS) * g_ref[...] + b_ref[...]
    u = jnp.maximum(u, 0.0)
    o = jnp.dot(u, Wc2_ref[...], preferred_element_type=jnp.float32)
    o_ref[...] = o + bc2_ref[...]


def _final(h1, h2, h3, W_jk, b_jk, Wc1, bc1, g, b, m, v, Wc2, bc2):
    row = pl.BlockSpec((BLK, H), lambda i: (i, 0))
    full = pl.BlockSpec((H, H), lambda i: (0, 0))
    vec = pl.BlockSpec((1, H), lambda i: (0, 0))
    return pl.pallas_call(
        _final_body,
        grid=(N // BLK,),
        in_specs=[row, row, row, full, full, full, vec, full, vec,
                  vec, vec, vec, vec, full, vec],
        out_specs=row,
        out_shape=jax.ShapeDtypeStruct((N, H), jnp.float32),
    )(h1, h2, h3, W_jk[0:H], W_jk[H:2 * H], W_jk[2 * H:3 * H],
      b_jk.reshape(1, H), Wc1, bc1.reshape(1, H), g.reshape(1, H),
      b.reshape(1, H), m.reshape(1, H), v.reshape(1, H), Wc2,
      bc2.reshape(1, H))


def kernel(x, edge_index, W1_0, b1_0, W2_0, b2_0, W1_1, b1_1, W2_1, b2_1,
           W1_2, b1_2, W2_2, b2_2, W_jk, b_jk, Wc1, bc1, bn_gamma, bn_beta,
           bn_mean, bn_var, Wc2, bc2):
    pad = jnp.arange(E_PAD - E, dtype=jnp.int32)
    src = jnp.concatenate([edge_index[0], pad % N]).reshape(NW * NCH, CH)
    dst = jnp.concatenate([edge_index[1], N + pad % (N_PAD - N)]
                          ).reshape(NW * NCH, CH)
    zeros = jnp.zeros((N_PAD, H), jnp.float32)
    Ws = [(W1_0, b1_0, W2_0, b2_0), (W1_1, b1_1, W2_1, b2_1),
          (W1_2, b1_2, W2_2, b2_2)]
    h = x
    xs = []
    for (W1, b1, W2, b2) in Ws:
        parts = _agg_call(h, src, dst, zeros)
        h = _mlp_layer(h, parts[0, :N], parts[1, :N], W1, b1, W2, b2)
        xs.append(h)
    return _final(xs[0], xs[1], xs[2], W_jk, b_jk, Wc1, bc1, bn_gamma,
                  bn_beta, bn_mean, bn_var, Wc2, bc2)
